# Initial kernel scaffold; baseline (speedup 1.0000x reference)
import jax
import jax.numpy as jnp
from jax.experimental import pallas as pl


def kernel(input_ids, token_table, position_table):
    raise NotImplementedError("write your pallas kernel here")


# speedup vs baseline: 1.1990x; 1.1990x over previous
import jax
import jax.numpy as jnp
from jax import lax
from jax.experimental import pallas as pl
from jax.experimental.pallas import tpu as pltpu
from jax.experimental.pallas import tpu_sc as plsc

VOCAB = 1000000
LENGTH = 200
EMBED = 32
BATCH = 4096

NW = 32
CHUNK = 128
TOKENS = BATCH * LENGTH
PER_W = TOKENS // NW
NCHUNK = PER_W // CHUNK
LANES = 16
NBT = BATCH // CHUNK
DT = EMBED // 8


def _emb_body(ids_hbm, pos_hbm, table_hbm, out_hbm,
              idx_v, pos_v, rows0, rows1, ot0, ot1, gs0, gs1, os0, os1):
    wid = lax.axis_index("s") * 2 + lax.axis_index("c")
    pltpu.sync_copy(ids_hbm.at[pl.ds(wid * NCHUNK, NCHUNK)], idx_v)
    pltpu.sync_copy(pos_hbm, pos_v)

    rows = (rows0, rows1)
    outb = (ot0, ot1)
    gsem = (gs0, gs1)
    osem = (os0, os1)

    iota16 = lax.iota(jnp.int32, 16)
    ridx = [iota16 + blq * LANES for blq in range(CHUNK // LANES)]

    def chunk_lbt(cc):
        g = wid * NCHUNK + cc
        l = lax.div(g, 8 * NBT) * 8 + lax.rem(g, 8)
        bt = lax.rem(lax.div(g, 8), NBT)
        return l, bt

    def out_dst(cc):
        l, bt = chunk_lbt(cc)
        return out_hbm.at[l, :, bt]

    pltpu.make_async_copy(table_hbm.at[idx_v.at[0]], rows0, gs0).start()

    def pair(i, carry):
        for b in range(2):
            cc = i * 2 + b
            nxt = 1 - b

            @pl.when(jnp.logical_and(cc >= 1, cc < NCHUNK - 1))
            def _drain():
                pltpu.make_async_copy(
                    outb[nxt], out_dst(cc - 1), osem[nxt]).wait()

            @pl.when(cc < NCHUNK - 1)
            def _prefetch():
                pltpu.make_async_copy(
                    table_hbm.at[idx_v.at[cc + 1]], rows[nxt],
                    gsem[nxt]).start()

            pltpu.make_async_copy(
                table_hbm.at[idx_v.at[cc]], rows[b], gsem[b]).wait()

            l, _ = chunk_lbt(cc)

            def sub_body(sub, c2):
                for dt in range(DT):
                    d = dt * 8 + sub
                    bd = jnp.full((LANES,), d, jnp.int32)
                    pval = plsc.load_gather(
                        pos_v, [jnp.full((LANES,), d * LENGTH + l, jnp.int32)])
                    for blq in range(CHUNK // LANES):
                        src = plsc.load_gather(rows[b], [ridx[blq], bd])
                        outb[b][dt, sub, pl.ds(blq * LANES, LANES)] = src + pval
                return c2

            lax.fori_loop(0, 8, sub_body, 0)

            pltpu.make_async_copy(outb[b], out_dst(cc), osem[b]).start()
        return carry

    lax.fori_loop(0, NCHUNK // 2, pair, 0)

    pltpu.make_async_copy(ot0, out_dst(NCHUNK - 2), os0).wait()
    pltpu.make_async_copy(ot1, out_dst(NCHUNK - 1), os1).wait()


@jax.jit
def _emb_call(ids, pos, table):
    mesh = plsc.VectorSubcoreMesh(core_axis_name="c", subcore_axis_name="s")
    f = pl.kernel(
        _emb_body,
        out_type=jax.ShapeDtypeStruct((LENGTH, DT, NBT, 8, CHUNK),
                                      jnp.float32),
        mesh=mesh,
        compiler_params=pltpu.CompilerParams(use_tc_tiling_on_sc=False,
                                             needs_layout_passes=False),
        scratch_types=[
            pltpu.VMEM((NCHUNK, CHUNK), jnp.int32),
            pltpu.VMEM((LENGTH * EMBED,), jnp.float32),
            pltpu.VMEM((CHUNK, EMBED), jnp.float32),
            pltpu.VMEM((CHUNK, EMBED), jnp.float32),
            pltpu.VMEM((DT, 8, CHUNK), jnp.float32),
            pltpu.VMEM((DT, 8, CHUNK), jnp.float32),
            pltpu.SemaphoreType.DMA,
            pltpu.SemaphoreType.DMA,
            pltpu.SemaphoreType.DMA,
            pltpu.SemaphoreType.DMA,
        ],
    )
    return f(ids, pos, table)


def kernel(input_ids, token_table, position_table):
    ids = (input_ids.astype(jnp.int32).T
           .reshape(LENGTH // 8, 8, NBT, CHUNK)
           .transpose(0, 2, 1, 3)
           .reshape(TOKENS // CHUNK, CHUNK))
    pos = position_table.T.reshape(LENGTH * EMBED)
    out5 = _emb_call(ids, pos, token_table)
    return out5.transpose(2, 4, 0, 1, 3).reshape(BATCH, LENGTH, EMBED)


# speedup vs baseline: 1.3514x; 1.1271x over previous
import jax
import jax.numpy as jnp
from jax import lax
from jax.experimental import pallas as pl
from jax.experimental.pallas import tpu as pltpu
from jax.experimental.pallas import tpu_sc as plsc

VOCAB = 1000000
LENGTH = 200
EMBED = 32
BATCH = 4096

NW = 32
CHUNK = 128
TOKENS = BATCH * LENGTH
PER_W = TOKENS // NW
NCHUNK = PER_W // CHUNK
LANES = 16
NBT = BATCH // CHUNK
DT = EMBED // 8


def _emb_body(ids_hbm, pos_hbm, table_hbm, out_hbm,
              idx_v, pos_v, rows0, rows1, ot0, ot1, gs0, gs1, os0, os1):
    wid = lax.axis_index("s") * 2 + lax.axis_index("c")
    pltpu.sync_copy(ids_hbm.at[pl.ds(wid * NCHUNK, NCHUNK)], idx_v)
    pltpu.sync_copy(pos_hbm, pos_v)

    rows = (rows0, rows1)
    outb = (ot0, ot1)
    gsem = (gs0, gs1)
    osem = (os0, os1)

    iota16 = lax.iota(jnp.int32, 16)
    idt = [lax.shift_right_logical(iota16 + h * LANES, 3) for h in range(2)]
    isub = [lax.bitwise_and(iota16 + h * LANES, 7) for h in range(2)]

    def chunk_lbt(cc):
        g = wid * NCHUNK + cc
        l = lax.div(g, 8 * NBT) * 8 + lax.rem(g, 8)
        bt = lax.rem(lax.div(g, 8), NBT)
        return l, bt

    def out_dst(cc):
        l, bt = chunk_lbt(cc)
        return out_hbm.at[l, :, bt]

    pltpu.make_async_copy(table_hbm.at[idx_v.at[0]], rows0, gs0).start()

    def pair(i, carry):
        for b in range(2):
            cc = i * 2 + b
            nxt = 1 - b

            @pl.when(jnp.logical_and(cc >= 1, cc < NCHUNK - 1))
            def _drain():
                pltpu.make_async_copy(
                    outb[nxt], out_dst(cc - 1), osem[nxt]).wait()

            @pl.when(cc < NCHUNK - 1)
            def _prefetch():
                pltpu.make_async_copy(
                    table_hbm.at[idx_v.at[cc + 1]], rows[nxt],
                    gsem[nxt]).start()

            pltpu.make_async_copy(
                table_hbm.at[idx_v.at[cc]], rows[b], gsem[b]).wait()

            l, _ = chunk_lbt(cc)
            pos_c = [pos_v[pl.ds(l * EMBED + h * LANES, LANES)]
                     for h in range(2)]

            def tok_body(jj, c2):
                for u in range(8):
                    j = jj * 8 + u
                    bj = jnp.full((LANES,), j, jnp.int32)
                    for h in range(2):
                        val = rows[b][j, pl.ds(h * LANES, LANES)] + pos_c[h]
                        plsc.store_scatter(outb[b], [idt[h], isub[h], bj], val)
                return c2

            lax.fori_loop(0, CHUNK // 8, tok_body, 0)

            pltpu.make_async_copy(outb[b], out_dst(cc), osem[b]).start()
        return carry

    lax.fori_loop(0, NCHUNK // 2, pair, 0)

    pltpu.make_async_copy(ot0, out_dst(NCHUNK - 2), os0).wait()
    pltpu.make_async_copy(ot1, out_dst(NCHUNK - 1), os1).wait()


@jax.jit
def _emb_call(ids, pos, table4):
    mesh = plsc.VectorSubcoreMesh(core_axis_name="c", subcore_axis_name="s")
    f = pl.kernel(
        _emb_body,
        out_type=jax.ShapeDtypeStruct((LENGTH, DT, NBT, 8, CHUNK),
                                      jnp.float32),
        mesh=mesh,
        compiler_params=pltpu.CompilerParams(use_tc_tiling_on_sc=False,
                                             needs_layout_passes=False),
        scratch_types=[
            pltpu.VMEM((NCHUNK, CHUNK), jnp.int32),
            pltpu.VMEM((LENGTH * EMBED,), jnp.float32),
            pltpu.VMEM((CHUNK, EMBED), jnp.float32),
            pltpu.VMEM((CHUNK, EMBED), jnp.float32),
            pltpu.VMEM((DT, 8, CHUNK), jnp.float32),
            pltpu.VMEM((DT, 8, CHUNK), jnp.float32),
            pltpu.SemaphoreType.DMA,
            pltpu.SemaphoreType.DMA,
            pltpu.SemaphoreType.DMA,
            pltpu.SemaphoreType.DMA,
        ],
    )
    return f(ids, pos, table4)


def kernel(input_ids, token_table, position_table):
    ids = ((input_ids.astype(jnp.int32) * 4).T
           .reshape(LENGTH // 8, 8, NBT, CHUNK)
           .transpose(0, 2, 1, 3)
           .reshape(TOKENS // CHUNK, CHUNK))
    pos = position_table.reshape(LENGTH * EMBED)
    table4 = jnp.pad(token_table, ((0, 0), (0, 96))).reshape(4 * VOCAB, EMBED)
    out5 = _emb_call(ids, pos, table4)
    return out5.transpose(2, 4, 0, 1, 3).reshape(BATCH, LENGTH, EMBED)
